# baseline (speedup 1.0000x reference)
import functools

import jax
import jax.numpy as jnp
from jax import lax
from jax.experimental import pallas as pl
from jax.experimental.pallas import tpu as pltpu
from jax.experimental.pallas import tpu_sc as plsc

N_NODES = 10000
N_EDGES = 320000
D_LAT = 128

NC = 2
NS = 16
NW = NC * NS
CH = 128
ACC_ROWS = 10240
NPS = ACC_ROWS // NS
_TSL = ACC_ROWS // NS
N_EPAD = 327680
HALF = N_EPAD // 2


def _dot(a, b):
    f32 = jnp.float32
    bf = jnp.bfloat16
    ah = a.astype(bf)
    al = (a - ah.astype(f32)).astype(bf)
    bh = b.astype(bf)
    bl = (b - bh.astype(f32)).astype(bf)

    def d(x, y):
        return jnp.dot(x, y, preferred_element_type=f32)

    return d(ah, bh) + d(ah, bl) + d(al, bh)


_sc_mesh = plsc.VectorSubcoreMesh(core_axis_name="c", subcore_axis_name="s")


def _make_gather(nrows):
    epc = nrows // NS
    nf = epc // CH
    assert nf % 4 == 0 and nf * CH == epc

    @functools.partial(
        pl.kernel,
        out_type=jax.ShapeDtypeStruct((NC, nrows, D_LAT), jnp.float32),
        mesh=_sc_mesh,
        scratch_types=[
            pltpu.VMEM((CH,), jnp.int32),
            pltpu.VMEM((CH,), jnp.int32),
            pltpu.VMEM((CH,), jnp.int32),
            pltpu.VMEM((CH,), jnp.int32),
            pltpu.VMEM((CH, D_LAT), jnp.float32),
            pltpu.VMEM((CH, D_LAT), jnp.float32),
            pltpu.VMEM_SHARED((ACC_ROWS, D_LAT), jnp.float32),
        ] + [pltpu.SemaphoreType.DMA] * 8,
    )
    def gather(vs_hbm, vd_hbm, idx_hbm, g_hbm,
               i0, i1, i2, i3, r0, r1, tab_sh,
               sa0, sa1, sa2, sa3, sb0, sb1, sc0, sc1):
        c = lax.axis_index("c")
        s = lax.axis_index("s")
        base = s * epc

        sl = pl.ds(s * _TSL, _TSL)

        @pl.when(c == 0)
        def _():
            pltpu.sync_copy(vs_hbm.at[sl], tab_sh.at[sl])

        @pl.when(c == 1)
        def _():
            pltpu.sync_copy(vd_hbm.at[sl], tab_sh.at[sl])

        plsc.subcore_barrier()

        idx = (i0, i1, i2, i3)
        rows = (r0, r1)
        sa = (sa0, sa1, sa2, sa3)
        sb = (sb0, sb1)
        sc = (sc0, sc1)

        def off(ci):
            return pl.ds(base + ci * CH, CH)

        def wait_a(ci, k):
            pltpu.make_async_copy(idx_hbm.at[c].at[off(ci)], idx[k], sa[k]).wait()

        def wait_b(k, b):
            pltpu.make_async_copy(tab_sh.at[idx[k]], rows[b], sb[b]).wait()

        def wait_c(ci, b):
            pltpu.make_async_copy(rows[b], g_hbm.at[c].at[off(ci)], sc[b]).wait()

        for k in range(4):
            pltpu.async_copy(idx_hbm.at[c].at[off(k)], idx[k], sa[k])

        @pl.loop(0, nf // 4)
        def _(g):
            for b4 in range(4):
                ci = g * 4 + b4
                b = b4 % 2
                wait_a(ci, b4)

                @pl.when(ci >= 2)
                def _():
                    wait_c(ci - 2, b)

                pltpu.async_copy(tab_sh.at[idx[b4]], rows[b], sb[b])

                @pl.when(ci >= 1)
                def _():
                    wait_b((b4 - 1) % 4, 1 - b)
                    pltpu.async_copy(rows[1 - b], g_hbm.at[c].at[off(ci - 1)], sc[1 - b])

                    @pl.when(ci + 3 < nf)
                    def _():
                        k = (b4 + 3) % 4
                        pltpu.async_copy(idx_hbm.at[c].at[off(ci + 3)], idx[k], sa[k])

        bl = (nf - 1) % 2
        wait_b((nf - 1) % 4, bl)
        pltpu.async_copy(rows[bl], g_hbm.at[c].at[off(nf - 1)], sc[bl])
        wait_c(nf - 2, 1 - bl)
        wait_c(nf - 1, bl)

    return gather


def _make_scatter(nrows):
    epw = nrows // NW
    nf = epw // CH
    assert nf % 2 == 0 and nf * CH == epw

    @functools.partial(
        pl.kernel,
        out_type=jax.ShapeDtypeStruct((NC, ACC_ROWS, D_LAT), jnp.float32),
        mesh=_sc_mesh,
        scratch_types=[
            pltpu.VMEM((CH,), jnp.int32),
            pltpu.VMEM((CH,), jnp.int32),
            pltpu.VMEM((CH, D_LAT), jnp.float32),
            pltpu.VMEM((CH, D_LAT), jnp.float32),
            pltpu.VMEM_SHARED((ACC_ROWS, D_LAT), jnp.float32),
        ] + [pltpu.SemaphoreType.DMA] * 4,
    )
    def scatter(e_hbm, dst_hbm, zeros_hbm, out_hbm,
                di0, di1, rows0, rows1, acc,
                sai0, sai1, sar0, sar1):
        c = lax.axis_index("c")
        s = lax.axis_index("s")
        wid = s * NC + c
        base = wid * epw

        pltpu.sync_copy(zeros_hbm.at[pl.ds(s * NPS, NPS)], acc.at[pl.ds(s * NPS, NPS)])
        plsc.subcore_barrier()

        di = (di0, di1)
        rows = (rows0, rows1)
        sai = (sai0, sai1)
        sar = (sar0, sar1)

        def off(ci):
            return pl.ds(base + ci * CH, CH)

        for b in (0, 1):
            pltpu.async_copy(dst_hbm.at[off(b)], di[b], sai[b])
            pltpu.async_copy(e_hbm.at[off(b)], rows[b], sar[b])

        @pl.loop(0, nf // 2)
        def _(g):
            for b in (0, 1):
                ci = g * 2 + b
                pltpu.make_async_copy(dst_hbm.at[off(ci)], di[b], sai[b]).wait()
                pltpu.make_async_copy(e_hbm.at[off(ci)], rows[b], sar[b]).wait()
                pltpu.sync_copy(rows[b], acc.at[di[b]], add=True)

                @pl.when(ci < nf - 2)
                def _():
                    pltpu.async_copy(dst_hbm.at[off(ci + 2)], di[b], sai[b])
                    pltpu.async_copy(e_hbm.at[off(ci + 2)], rows[b], sar[b])

        plsc.subcore_barrier()
        pltpu.sync_copy(acc.at[pl.ds(s * NPS, NPS)], out_hbm.at[c].at[pl.ds(s * NPS, NPS)])

    return scatter


_gather_half = _make_gather(HALF)
_scatter_half = _make_scatter(HALF)
_gather_full = _make_gather(N_EPAD)
_scatter_full = _make_scatter(N_EPAD)


_R_NODE = 2000
_R_EDGE = 8192


def _wspec(r, c):
    return pl.BlockSpec((r, c), lambda i: (0, 0))


def _rspec(r, c):
    return pl.BlockSpec((r, c), lambda i: (i, 0))


def _node_encode_body(x_ref, w1, b1, w2, b2, ws, wd, v_ref, vs_ref, vd_ref):
    h = _dot(x_ref[...], w1[...]) + b1[...]
    v = _dot(h, w2[...]) + b2[...]
    v_ref[...] = v
    vs_ref[...] = _dot(v, ws[...])
    vd_ref[...] = _dot(v, wd[...])


def _gsum(gs_ref, gd_ref):
    return gs_ref[...].reshape(gs_ref.shape[1:]) + gd_ref[...].reshape(gd_ref.shape[1:])


def _edge_step1_body(ef_ref, gs_ref, gd_ref, we1, be1, we2, be2,
                     w1e, b1, w2, b2, out_ref):
    e0 = _dot(ef_ref[...], we1[...]) + be1[...]
    e0 = _dot(e0, we2[...]) + be2[...]
    g = _gsum(gs_ref, gd_ref)
    h = _dot(e0, w1e[...]) + g + b1[...]
    out_ref[...] = e0 + _dot(h, w2[...]) + b2[...]


def _edge_step2_body(e_ref, gs_ref, gd_ref, w1e, b1, w2, b2, out_ref):
    g = _gsum(gs_ref, gd_ref)
    h = _dot(e_ref[...], w1e[...]) + g + b1[...]
    out_ref[...] = e_ref[...] + _dot(h, w2[...]) + b2[...]


def _agg4(p0_ref, p1_ref, p2_ref, p3_ref):
    return p0_ref[...] + p1_ref[...] + p2_ref[...] + p3_ref[...]


def _node_update_body(v_ref, p0, p1, w1v, w1a, b1, w2, b2, ws, wd,
                      v1_ref, vs_ref, vd_ref):
    agg = p0[...] + p1[...]
    h = _dot(v_ref[...], w1v[...]) + _dot(agg, w1a[...]) + b1[...]
    v1 = v_ref[...] + _dot(h, w2[...]) + b2[...]
    v1_ref[...] = v1
    vs_ref[...] = _dot(v1, ws[...])
    vd_ref[...] = _dot(v1, wd[...])


def _node_final_body(v_ref, p0, p1, w1v, w1a, b1, w2, b2,
                     d1, db1, d2, db2, out_ref):
    agg = p0[...] + p1[...]
    h = _dot(v_ref[...], w1v[...]) + _dot(agg, w1a[...]) + b1[...]
    v2 = v_ref[...] + _dot(h, w2[...]) + b2[...]
    o = _dot(v2, d1[...]) + db1[...]
    out_ref[...] = _dot(o, d2[...]) + db2[...]


def _tc_call(body, grid, in_specs, out_specs, out_shapes, *args):
    return pl.pallas_call(
        body,
        grid=(grid,),
        in_specs=in_specs,
        out_specs=out_specs,
        out_shape=out_shapes,
        compiler_params=pltpu.CompilerParams(
            dimension_semantics=("parallel",)),
    )(*args)


def kernel(node_features_in, edges_indexes, edge_features_in, params):
    f32 = jnp.float32
    npad = N_EPAD - N_EDGES
    src_pad = jnp.concatenate(
        [edges_indexes[0], jnp.zeros((npad,), jnp.int32)])
    dst_pad = jnp.concatenate(
        [edges_indexes[1],
         N_NODES + (jnp.arange(npad, dtype=jnp.int32) % (ACC_ROWS - N_NODES))])
    ei = jnp.stack([src_pad, dst_pad])
    ef_pad = jnp.pad(edge_features_in, ((0, npad), (0, 0)))

    def _wb(layer):
        return layer["W"], layer["b"].reshape(1, -1)

    enW1, enb1 = _wb(params["enc_node"][0])
    enW2, enb2 = _wb(params["enc_node"][1])
    eeW1, eeb1 = _wb(params["enc_edge"][0])
    eeW2, eeb2 = _wb(params["enc_edge"][1])
    dW1, db1 = _wb(params["dec"][0])
    dW2, db2 = _wb(params["dec"][1])

    steps = []
    for t in range(2):
        pe = params["proc"][t]["edge"]
        pn = params["proc"][t]["node"]
        W1, b1 = _wb(pe[0])
        W2, b2 = _wb(pe[1])
        nW1, nb1 = _wb(pn[0])
        nW2, nb2 = _wb(pn[1])
        steps.append(dict(
            W1e=W1[:D_LAT], W1s=W1[D_LAT:2 * D_LAT], W1d=W1[2 * D_LAT:],
            b1=b1, W2=W2, b2=b2,
            nW1v=nW1[:D_LAT], nW1a=nW1[D_LAT:], nb1=nb1, nW2=nW2, nb2=nb2,
        ))

    zeros_nodes = jnp.zeros((ACC_ROWS, D_LAT), f32)

    nb = N_NODES // _R_NODE
    w128 = _wspec(D_LAT, D_LAT)
    bia = _wspec(1, D_LAT)
    nrow = _rspec(_R_NODE, D_LAT)
    nshape = jax.ShapeDtypeStruct((N_NODES, D_LAT), f32)

    eb = N_EPAD // _R_EDGE
    erow = _rspec(_R_EDGE, D_LAT)
    eshape = jax.ShapeDtypeStruct((N_EPAD, D_LAT), f32)
    g0spec = pl.BlockSpec((1, _R_EDGE, D_LAT), lambda i: (0, i, 0))
    g1spec = pl.BlockSpec((1, _R_EDGE, D_LAT), lambda i: (1, i, 0))

    def _tab(x):
        return jnp.pad(x, ((0, ACC_ROWS - N_NODES), (0, 0)))

    def edge_step1(ef_h, g_h, st):
        return _tc_call(
            _edge_step1_body, eb,
            [_rspec(_R_EDGE, 16), g0spec, g1spec,
             _wspec(16, D_LAT), bia, w128, bia, w128, bia, w128, bia],
            erow, eshape,
            ef_h, g_h, g_h, eeW1, eeb1, eeW2, eeb2,
            st["W1e"], st["b1"], st["W2"], st["b2"])

    def edge_step2(e_h, g_h, st):
        return _tc_call(
            _edge_step2_body, eb,
            [erow, g0spec, g1spec, w128, bia, w128, bia],
            erow, eshape,
            e_h, g_h, g_h,
            st["W1e"], st["b1"], st["W2"], st["b2"])

    v0, vs1, vd1 = _tc_call(
        _node_encode_body, nb,
        [nrow, w128, bia, w128, bia, w128, w128],
        [nrow, nrow, nrow], [nshape, nshape, nshape],
        node_features_in, enW1, enb1, enW2, enb2,
        steps[0]["W1s"], steps[0]["W1d"])

    t1s, t1d = _tab(vs1), _tab(vd1)
    g1 = _gather_full(t1s, t1d, ei)
    e1 = edge_step1(ef_pad, g1, steps[0])
    p1 = _scatter_full(e1, dst_pad, zeros_nodes)

    v1, vs2, vd2 = _tc_call(
        _node_update_body, nb,
        [nrow, nrow, nrow, w128, w128, bia, w128, bia, w128, w128],
        [nrow, nrow, nrow], [nshape, nshape, nshape],
        v0, p1[0, :N_NODES], p1[1, :N_NODES],
        steps[0]["nW1v"], steps[0]["nW1a"], steps[0]["nb1"],
        steps[0]["nW2"], steps[0]["nb2"],
        steps[1]["W1s"], steps[1]["W1d"])

    t2s, t2d = _tab(vs2), _tab(vd2)
    g2 = _gather_full(t2s, t2d, ei)
    e2 = edge_step2(e1, g2, steps[1])
    p2 = _scatter_full(e2, dst_pad, zeros_nodes)

    out = _tc_call(
        _node_final_body, nb,
        [nrow, nrow, nrow, w128, w128, bia, w128, bia,
         w128, bia, w128, bia],
        nrow, nshape,
        v1, p2[0, :N_NODES], p2[1, :N_NODES],
        steps[1]["nW1v"], steps[1]["nW1a"], steps[1]["nb1"],
        steps[1]["nW2"], steps[1]["nb2"],
        dW1, db1, dW2, db2)

    return out


# speedup vs baseline: 1.0958x; 1.0958x over previous
import functools

import jax
import jax.numpy as jnp
from jax import lax
from jax.experimental import pallas as pl
from jax.experimental.pallas import tpu as pltpu
from jax.experimental.pallas import tpu_sc as plsc

N_NODES = 10000
N_EDGES = 320000
D_LAT = 128

NC = 2
NS = 16
NW = NC * NS
CH = 128
ACC_ROWS = 10240
NPS = ACC_ROWS // NS
_TSL = ACC_ROWS // NS
N_EPAD = 327680
HALF = N_EPAD // 2


def _dot(a, b):
    f32 = jnp.float32
    bf = jnp.bfloat16
    ah = a.astype(bf)
    al = (a - ah.astype(f32)).astype(bf)
    bh = b.astype(bf)
    bl = (b - bh.astype(f32)).astype(bf)

    def d(x, y):
        return jnp.dot(x, y, preferred_element_type=f32)

    return d(ah, bh) + d(ah, bl) + d(al, bh)


_sc_mesh = plsc.VectorSubcoreMesh(core_axis_name="c", subcore_axis_name="s")


def _make_gather(nrows):
    epc = nrows // NS
    nf = epc // CH
    assert nf % 4 == 0 and nf * CH == epc

    @functools.partial(
        pl.kernel,
        out_type=jax.ShapeDtypeStruct((NC, nrows, D_LAT), jnp.float32),
        mesh=_sc_mesh,
        scratch_types=[
            pltpu.VMEM((CH,), jnp.int32),
            pltpu.VMEM((CH,), jnp.int32),
            pltpu.VMEM((CH,), jnp.int32),
            pltpu.VMEM((CH,), jnp.int32),
            pltpu.VMEM((CH, D_LAT), jnp.float32),
            pltpu.VMEM((CH, D_LAT), jnp.float32),
            pltpu.VMEM_SHARED((ACC_ROWS, D_LAT), jnp.float32),
        ] + [pltpu.SemaphoreType.DMA] * 8,
    )
    def gather(vs_hbm, vd_hbm, idx_hbm, g_hbm,
               i0, i1, i2, i3, r0, r1, tab_sh,
               sa0, sa1, sa2, sa3, sb0, sb1, sc0, sc1):
        c = lax.axis_index("c")
        s = lax.axis_index("s")
        base = s * epc

        sl = pl.ds(s * _TSL, _TSL)

        @pl.when(c == 0)
        def _():
            pltpu.sync_copy(vs_hbm.at[sl], tab_sh.at[sl])

        @pl.when(c == 1)
        def _():
            pltpu.sync_copy(vd_hbm.at[sl], tab_sh.at[sl])

        plsc.subcore_barrier()

        idx = (i0, i1, i2, i3)
        rows = (r0, r1)
        sa = (sa0, sa1, sa2, sa3)
        sb = (sb0, sb1)
        sc = (sc0, sc1)

        def off(ci):
            return pl.ds(base + ci * CH, CH)

        def wait_a(ci, k):
            pltpu.make_async_copy(idx_hbm.at[c].at[off(ci)], idx[k], sa[k]).wait()

        def wait_b(k, b):
            pltpu.make_async_copy(tab_sh.at[idx[k]], rows[b], sb[b]).wait()

        def wait_c(ci, b):
            pltpu.make_async_copy(rows[b], g_hbm.at[c].at[off(ci)], sc[b]).wait()

        for k in range(4):
            pltpu.async_copy(idx_hbm.at[c].at[off(k)], idx[k], sa[k])

        @pl.loop(0, nf // 4)
        def _(g):
            for b4 in range(4):
                ci = g * 4 + b4
                b = b4 % 2
                wait_a(ci, b4)

                @pl.when(ci >= 2)
                def _():
                    wait_c(ci - 2, b)

                pltpu.async_copy(tab_sh.at[idx[b4]], rows[b], sb[b])

                @pl.when(ci >= 1)
                def _():
                    wait_b((b4 - 1) % 4, 1 - b)
                    pltpu.async_copy(rows[1 - b], g_hbm.at[c].at[off(ci - 1)], sc[1 - b])

                    @pl.when(ci + 3 < nf)
                    def _():
                        k = (b4 + 3) % 4
                        pltpu.async_copy(idx_hbm.at[c].at[off(ci + 3)], idx[k], sa[k])

        bl = (nf - 1) % 2
        wait_b((nf - 1) % 4, bl)
        pltpu.async_copy(rows[bl], g_hbm.at[c].at[off(nf - 1)], sc[bl])
        wait_c(nf - 2, 1 - bl)
        wait_c(nf - 1, bl)

    return gather


def _make_scatter(nrows):
    epw = nrows // NW
    nf = epw // CH
    assert nf % 2 == 0 and nf * CH == epw

    @functools.partial(
        pl.kernel,
        out_type=jax.ShapeDtypeStruct((NC, ACC_ROWS, D_LAT), jnp.float32),
        mesh=_sc_mesh,
        scratch_types=[
            pltpu.VMEM((CH,), jnp.int32),
            pltpu.VMEM((CH,), jnp.int32),
            pltpu.VMEM((CH, D_LAT), jnp.float32),
            pltpu.VMEM((CH, D_LAT), jnp.float32),
            pltpu.VMEM_SHARED((ACC_ROWS, D_LAT), jnp.float32),
        ] + [pltpu.SemaphoreType.DMA] * 4,
    )
    def scatter(e_hbm, dst_hbm, init_hbm, out_hbm,
                di0, di1, rows0, rows1, acc,
                sai0, sai1, sar0, sar1):
        c = lax.axis_index("c")
        s = lax.axis_index("s")
        wid = s * NC + c
        base = wid * epw

        pltpu.sync_copy(init_hbm.at[c].at[pl.ds(s * NPS, NPS)], acc.at[pl.ds(s * NPS, NPS)])
        plsc.subcore_barrier()

        di = (di0, di1)
        rows = (rows0, rows1)
        sai = (sai0, sai1)
        sar = (sar0, sar1)

        def off(ci):
            return pl.ds(base + ci * CH, CH)

        for b in (0, 1):
            pltpu.async_copy(dst_hbm.at[off(b)], di[b], sai[b])
            pltpu.async_copy(e_hbm.at[off(b)], rows[b], sar[b])

        @pl.loop(0, nf // 2)
        def _(g):
            for b in (0, 1):
                ci = g * 2 + b
                pltpu.make_async_copy(dst_hbm.at[off(ci)], di[b], sai[b]).wait()
                pltpu.make_async_copy(e_hbm.at[off(ci)], rows[b], sar[b]).wait()
                pltpu.sync_copy(rows[b], acc.at[di[b]], add=True)

                @pl.when(ci < nf - 2)
                def _():
                    pltpu.async_copy(dst_hbm.at[off(ci + 2)], di[b], sai[b])
                    pltpu.async_copy(e_hbm.at[off(ci + 2)], rows[b], sar[b])

        plsc.subcore_barrier()
        pltpu.sync_copy(acc.at[pl.ds(s * NPS, NPS)], out_hbm.at[c].at[pl.ds(s * NPS, NPS)])

    return scatter


_gather_half = _make_gather(HALF)
_scatter_half = _make_scatter(HALF)
QUARTER = N_EPAD // 4
_gather_q = _make_gather(QUARTER)
_scatter_q = _make_scatter(QUARTER)


_R_NODE = 2000
_R_EDGE = 8192


def _wspec(r, c):
    return pl.BlockSpec((r, c), lambda i: (0, 0))


def _rspec(r, c):
    return pl.BlockSpec((r, c), lambda i: (i, 0))


def _node_encode_body(x_ref, w1, b1, w2, b2, ws, wd, v_ref, vs_ref, vd_ref):
    h = _dot(x_ref[...], w1[...]) + b1[...]
    v = _dot(h, w2[...]) + b2[...]
    v_ref[...] = v
    vs_ref[...] = _dot(v, ws[...])
    vd_ref[...] = _dot(v, wd[...])


def _gsum(gs_ref, gd_ref):
    return gs_ref[...].reshape(gs_ref.shape[1:]) + gd_ref[...].reshape(gd_ref.shape[1:])


def _edge_step1_body(ef_ref, gs_ref, gd_ref, we1, be1, we2, be2,
                     w1e, b1, w2, b2, out_ref):
    e0 = _dot(ef_ref[...], we1[...]) + be1[...]
    e0 = _dot(e0, we2[...]) + be2[...]
    g = _gsum(gs_ref, gd_ref)
    h = _dot(e0, w1e[...]) + g + b1[...]
    out_ref[...] = e0 + _dot(h, w2[...]) + b2[...]


def _edge_step2_body(e_ref, gs_ref, gd_ref, w1e, b1, w2, b2, out_ref):
    g = _gsum(gs_ref, gd_ref)
    h = _dot(e_ref[...], w1e[...]) + g + b1[...]
    out_ref[...] = e_ref[...] + _dot(h, w2[...]) + b2[...]


def _agg4(p0_ref, p1_ref, p2_ref, p3_ref):
    return p0_ref[...] + p1_ref[...] + p2_ref[...] + p3_ref[...]


def _node_update_body(v_ref, p0, p1, w1v, w1a, b1, w2, b2, ws, wd,
                      v1_ref, vs_ref, vd_ref):
    agg = p0[...] + p1[...]
    h = _dot(v_ref[...], w1v[...]) + _dot(agg, w1a[...]) + b1[...]
    v1 = v_ref[...] + _dot(h, w2[...]) + b2[...]
    v1_ref[...] = v1
    vs_ref[...] = _dot(v1, ws[...])
    vd_ref[...] = _dot(v1, wd[...])


def _node_final_body(v_ref, p0, p1, w1v, w1a, b1, w2, b2,
                     d1, db1, d2, db2, out_ref):
    agg = p0[...] + p1[...]
    h = _dot(v_ref[...], w1v[...]) + _dot(agg, w1a[...]) + b1[...]
    v2 = v_ref[...] + _dot(h, w2[...]) + b2[...]
    o = _dot(v2, d1[...]) + db1[...]
    out_ref[...] = _dot(o, d2[...]) + db2[...]


def _tc_call(body, grid, in_specs, out_specs, out_shapes, *args):
    return pl.pallas_call(
        body,
        grid=(grid,),
        in_specs=in_specs,
        out_specs=out_specs,
        out_shape=out_shapes,
        compiler_params=pltpu.CompilerParams(
            dimension_semantics=("parallel",)),
    )(*args)


def kernel(node_features_in, edges_indexes, edge_features_in, params):
    f32 = jnp.float32
    npad = N_EPAD - N_EDGES
    src_pad = jnp.concatenate(
        [edges_indexes[0], jnp.zeros((npad,), jnp.int32)])
    dst_pad = jnp.concatenate(
        [edges_indexes[1],
         N_NODES + (jnp.arange(npad, dtype=jnp.int32) % (ACC_ROWS - N_NODES))])
    ei = jnp.stack([src_pad, dst_pad])
    ef_pad = jnp.pad(edge_features_in, ((0, npad), (0, 0)))
    qs = [slice(q * QUARTER, (q + 1) * QUARTER) for q in range(4)]
    ei_q = [ei[:, sl] for sl in qs]
    dst_q = [dst_pad[sl] for sl in qs]
    ef_q = [ef_pad[sl] for sl in qs]

    def _wb(layer):
        return layer["W"], layer["b"].reshape(1, -1)

    enW1, enb1 = _wb(params["enc_node"][0])
    enW2, enb2 = _wb(params["enc_node"][1])
    eeW1, eeb1 = _wb(params["enc_edge"][0])
    eeW2, eeb2 = _wb(params["enc_edge"][1])
    dW1, db1 = _wb(params["dec"][0])
    dW2, db2 = _wb(params["dec"][1])

    steps = []
    for t in range(2):
        pe = params["proc"][t]["edge"]
        pn = params["proc"][t]["node"]
        W1, b1 = _wb(pe[0])
        W2, b2 = _wb(pe[1])
        nW1, nb1 = _wb(pn[0])
        nW2, nb2 = _wb(pn[1])
        steps.append(dict(
            W1e=W1[:D_LAT], W1s=W1[D_LAT:2 * D_LAT], W1d=W1[2 * D_LAT:],
            b1=b1, W2=W2, b2=b2,
            nW1v=nW1[:D_LAT], nW1a=nW1[D_LAT:], nb1=nb1, nW2=nW2, nb2=nb2,
        ))

    zeros_nodes = jnp.zeros((NC, ACC_ROWS, D_LAT), f32)

    nb = N_NODES // _R_NODE
    w128 = _wspec(D_LAT, D_LAT)
    bia = _wspec(1, D_LAT)
    nrow = _rspec(_R_NODE, D_LAT)
    nshape = jax.ShapeDtypeStruct((N_NODES, D_LAT), f32)

    eb = QUARTER // _R_EDGE
    erow = _rspec(_R_EDGE, D_LAT)
    eshape = jax.ShapeDtypeStruct((QUARTER, D_LAT), f32)
    g0spec = pl.BlockSpec((1, _R_EDGE, D_LAT), lambda i: (0, i, 0))
    g1spec = pl.BlockSpec((1, _R_EDGE, D_LAT), lambda i: (1, i, 0))

    def _tab(x):
        return jnp.pad(x, ((0, ACC_ROWS - N_NODES), (0, 0)))

    def edge_step1(ef_h, g_h, st):
        return _tc_call(
            _edge_step1_body, eb,
            [_rspec(_R_EDGE, 16), g0spec, g1spec,
             _wspec(16, D_LAT), bia, w128, bia, w128, bia, w128, bia],
            erow, eshape,
            ef_h, g_h, g_h, eeW1, eeb1, eeW2, eeb2,
            st["W1e"], st["b1"], st["W2"], st["b2"])

    def edge_step2(e_h, g_h, st):
        return _tc_call(
            _edge_step2_body, eb,
            [erow, g0spec, g1spec, w128, bia, w128, bia],
            erow, eshape,
            e_h, g_h, g_h,
            st["W1e"], st["b1"], st["W2"], st["b2"])

    v0, vs1, vd1 = _tc_call(
        _node_encode_body, nb,
        [nrow, w128, bia, w128, bia, w128, w128],
        [nrow, nrow, nrow], [nshape, nshape, nshape],
        node_features_in, enW1, enb1, enW2, enb2,
        steps[0]["W1s"], steps[0]["W1d"])

    t1s, t1d = _tab(vs1), _tab(vd1)
    g1 = [_gather_q(t1s, t1d, ei_q[q]) for q in range(4)]
    e1 = [edge_step1(ef_q[q], g1[q], steps[0]) for q in range(4)]
    p1 = zeros_nodes
    for q in range(4):
        p1 = _scatter_q(e1[q], dst_q[q], p1)

    v1, vs2, vd2 = _tc_call(
        _node_update_body, nb,
        [nrow, nrow, nrow, w128, w128, bia, w128, bia, w128, w128],
        [nrow, nrow, nrow], [nshape, nshape, nshape],
        v0, p1[0, :N_NODES], p1[1, :N_NODES],
        steps[0]["nW1v"], steps[0]["nW1a"], steps[0]["nb1"],
        steps[0]["nW2"], steps[0]["nb2"],
        steps[1]["W1s"], steps[1]["W1d"])

    t2s, t2d = _tab(vs2), _tab(vd2)
    g2 = [_gather_q(t2s, t2d, ei_q[q]) for q in range(4)]
    e2 = [edge_step2(e1[q], g2[q], steps[1]) for q in range(4)]
    p2 = zeros_nodes
    for q in range(4):
        p2 = _scatter_q(e2[q], dst_q[q], p2)

    out = _tc_call(
        _node_final_body, nb,
        [nrow, nrow, nrow, w128, w128, bia, w128, bia,
         w128, bia, w128, bia],
        nrow, nshape,
        v1, p2[0, :N_NODES], p2[1, :N_NODES],
        steps[1]["nW1v"], steps[1]["nW1a"], steps[1]["nb1"],
        steps[1]["nW2"], steps[1]["nb2"],
        dW1, db1, dW2, db2)

    return out
